# Initial kernel scaffold; baseline (speedup 1.0000x reference)
import jax
import jax.numpy as jnp
from jax.experimental import pallas as pl


def kernel(x, edge_index, Wr0, Wroot0, b0, Wr1, Wroot1, b1, Wr2, Wroot2, b2):
    raise NotImplementedError("write your pallas kernel here")


# speedup vs baseline: 4.5091x; 4.5091x over previous
import functools

import jax
import jax.numpy as jnp
from jax import lax
from jax.experimental import pallas as pl
from jax.experimental.pallas import tpu as pltpu
from jax.experimental.pallas import tpu_sc as plsc

N = 10000
E = 320000
D = 128

NC = 2
NS = 16
NPAD = 10240
CH = 80
EDGES_PER_TILE = E // (NC * NS)
NCHUNK = EDGES_PER_TILE // CH
ROWS_PER_TILE = NPAD // NS
ZROWS = 128
NSTRIPE = ROWS_PER_TILE // ZROWS


def _seg_sum_partials(h, src, dst):
    mesh = plsc.VectorSubcoreMesh(core_axis_name="c", subcore_axis_name="s")

    @functools.partial(
        pl.kernel,
        mesh=mesh,
        out_type=jax.ShapeDtypeStruct((NC, NPAD, D), jnp.float32),
        scratch_types=[
            pltpu.VMEM((CH,), jnp.int32),
            pltpu.VMEM((CH,), jnp.int32),
            pltpu.VMEM((CH, D), jnp.float32),
            pltpu.VMEM((ZROWS, D), jnp.float32),
            pltpu.VMEM_SHARED((NPAD, D), jnp.float32),
            pltpu.SemaphoreType.DMA,
        ],
    )
    def k(h_hbm, src_hbm, dst_hbm, out_hbm, src_v, dst_v, rows_v, zero_v, acc, sem):
        c = lax.axis_index("c")
        s = lax.axis_index("s")

        def zfill(t, carry):
            i = t // (D // 16)
            j = t % (D // 16)
            zero_v[i, pl.ds(j * 16, 16)] = jnp.zeros((16,), jnp.float32)
            return carry

        lax.fori_loop(0, ZROWS * (D // 16), zfill, 0)

        row0 = s * ROWS_PER_TILE
        for t in range(NSTRIPE):
            pltpu.sync_copy(zero_v, acc.at[pl.ds(row0 + t * ZROWS, ZROWS)])
        plsc.subcore_barrier()

        base = c * (E // NC) + s * EDGES_PER_TILE

        def body(j, carry):
            off = base + j * CH
            pltpu.sync_copy(src_hbm.at[pl.ds(off, CH)], src_v)
            pltpu.async_copy(h_hbm.at[src_v], rows_v, sem).wait()
            pltpu.sync_copy(dst_hbm.at[pl.ds(off, CH)], dst_v)
            pltpu.sync_copy(rows_v, acc.at[dst_v], add=True)
            return carry

        lax.fori_loop(0, NCHUNK, body, 0)
        plsc.subcore_barrier()

        for t in range(NSTRIPE):
            r = row0 + t * ZROWS
            pltpu.sync_copy(acc.at[pl.ds(r, ZROWS)], out_hbm.at[c, pl.ds(r, ZROWS)])

    return k(h, src, dst)


def _combine(p, h, Wr, Wroot, b2d, relu):
    BR = 640

    def body(p_ref, h_ref, wr_ref, wroot_ref, b_ref, o_ref):
        agg = p_ref[0] + p_ref[1]
        out = jnp.dot(agg, wr_ref[...], preferred_element_type=jnp.float32)
        out = out + jnp.dot(h_ref[...], wroot_ref[...],
                            preferred_element_type=jnp.float32)
        out = out + b_ref[...]
        if relu:
            out = jnp.maximum(out, 0.0)
        o_ref[...] = out

    return pl.pallas_call(
        body,
        grid=(NPAD // BR,),
        in_specs=[
            pl.BlockSpec((NC, BR, D), lambda i: (0, i, 0)),
            pl.BlockSpec((BR, D), lambda i: (i, 0)),
            pl.BlockSpec((D, D), lambda i: (0, 0)),
            pl.BlockSpec((D, D), lambda i: (0, 0)),
            pl.BlockSpec((1, D), lambda i: (0, 0)),
        ],
        out_specs=pl.BlockSpec((BR, D), lambda i: (i, 0)),
        out_shape=jax.ShapeDtypeStruct((NPAD, D), jnp.float32),
    )(p, h, Wr, Wroot, b2d)


def kernel(x, edge_index, Wr0, Wroot0, b0, Wr1, Wroot1, b1, Wr2, Wroot2, b2):
    src = edge_index[0].astype(jnp.int32)
    dst = edge_index[1].astype(jnp.int32)
    h = jnp.concatenate([x, jnp.zeros((NPAD - N, D), jnp.float32)], axis=0)
    for Wr, Wroot, b, relu in ((Wr0, Wroot0, b0, True),
                               (Wr1, Wroot1, b1, True),
                               (Wr2, Wroot2, b2, False)):
        p = _seg_sum_partials(h, src, dst)
        h = _combine(p, h, Wr, Wroot, b.reshape(1, D), relu)
    return h[:N]


# speedup vs baseline: 11.9102x; 2.6414x over previous
import functools

import jax
import jax.numpy as jnp
from jax import lax
from jax.experimental import pallas as pl
from jax.experimental.pallas import tpu as pltpu
from jax.experimental.pallas import tpu_sc as plsc

N = 10000
E = 320000
D = 128

NC = 2
NS = 16
NPAD = 10240
CH = 80
EDGES_PER_TILE = E // (NC * NS)
NCHUNK = EDGES_PER_TILE // CH
ROWS_PER_TILE = NPAD // NS
ZROWS = 128
NSTRIPE = ROWS_PER_TILE // ZROWS

NBUF = 4
NIDX = 8
GRP = 8
NGRP = 15


def _seg_sum_partials(h, src, dst):
    mesh = plsc.VectorSubcoreMesh(core_axis_name="c", subcore_axis_name="s")

    @functools.partial(
        pl.kernel,
        mesh=mesh,
        out_type=jax.ShapeDtypeStruct((NC, NPAD, D), jnp.float32),
        scratch_types=(
            [pltpu.VMEM((CH, D), jnp.float32) for _ in range(NBUF)]
            + [pltpu.VMEM((CH,), jnp.int32) for _ in range(NIDX)]
            + [pltpu.VMEM((CH,), jnp.int32) for _ in range(NIDX)]
            + [pltpu.VMEM_SHARED((NPAD, D), jnp.float32)]
            + [pltpu.SemaphoreType.DMA] * (2 * NBUF + 2 * NIDX)
        ),
    )
    def k(h_hbm, src_hbm, dst_hbm, out_hbm, *refs):
        rows = refs[0:NBUF]
        si = refs[NBUF:NBUF + NIDX]
        di = refs[NBUF + NIDX:NBUF + 2 * NIDX]
        acc = refs[NBUF + 2 * NIDX]
        sems = refs[NBUF + 2 * NIDX + 1:]
        gsem = sems[0:NBUF]
        ssem = sems[NBUF:2 * NBUF]
        sisem = sems[2 * NBUF:2 * NBUF + NIDX]
        disem = sems[2 * NBUF + NIDX:]

        c = lax.axis_index("c")
        s = lax.axis_index("s")
        base = c * (E // NC) + s * EDGES_PER_TILE

        def fetch_idx(j, q):
            off = base + j * CH
            pltpu.async_copy(src_hbm.at[pl.ds(off, CH)], si[q], sisem[q])
            pltpu.async_copy(dst_hbm.at[pl.ds(off, CH)], di[q], disem[q])

        def zfill(t, carry):
            i = t // (D // 16)
            j = t % (D // 16)
            rows[0][i, pl.ds(j * 16, 16)] = jnp.zeros((16,), jnp.float32)
            return carry

        lax.fori_loop(0, CH * (D // 16), zfill, 0)

        row0 = s * ROWS_PER_TILE
        for t in range(ROWS_PER_TILE // CH):
            pltpu.sync_copy(rows[0], acc.at[pl.ds(row0 + t * CH, CH)])
        plsc.subcore_barrier()

        for j in range(4):
            fetch_idx(j, j)
        for j in range(2):
            pltpu.make_async_copy(
                src_hbm.at[pl.ds(base + j * CH, CH)], si[j], sisem[j]).wait()
            pltpu.async_copy(h_hbm.at[si[j]], rows[j], gsem[j])

        def chunk_body(g, b, j):
            rb = b % NBUF
            rb2 = (b + 2) % NBUF
            q = b % NIDX
            q2 = (b + 2) % NIDX
            q4 = (b + 4) % NIDX
            qp = (b + NIDX - 2) % NIDX

            def fire_fetch():
                fetch_idx(j + 4, q4)

            def wait_scatter_prev():
                pltpu.make_async_copy(
                    rows[rb2], acc.at[di[qp]], ssem[rb2]).wait()

            def fire_gather_next():
                pltpu.make_async_copy(
                    src_hbm.at[pl.ds(base + (j + 2) * CH, CH)],
                    si[q2], sisem[q2]).wait()
                pltpu.async_copy(h_hbm.at[si[q2]], rows[rb2], gsem[q2 % NBUF])

            return fire_fetch, wait_scatter_prev, fire_gather_next, rb, q

        def emit_steady(g, b):
            j = g * GRP + b
            fire_fetch, wait_sc, fire_g, rb, q = chunk_body(g, b, j)
            if isinstance(j, int):
                if j + 4 < NCHUNK:
                    fire_fetch()
                wait_sc()
                if j + 2 < NCHUNK:
                    fire_g()
            else:
                fire_fetch()
                if b < 2:
                    @pl.when(j >= 2)
                    def _():
                        wait_sc()
                else:
                    wait_sc()
                fire_g()
            pltpu.make_async_copy(
                h_hbm.at[si[q]], rows[rb], gsem[rb]).wait()
            pltpu.make_async_copy(
                dst_hbm.at[pl.ds(base + j * CH, CH)], di[q], disem[q]).wait()
            pltpu.async_copy(rows[rb], acc.at[di[q]], ssem[rb], add=True)

        def group(g, carry):
            for b in range(GRP):
                emit_steady(g, b)
            return carry

        lax.fori_loop(0, NGRP, group, 0)

        for j in range(NGRP * GRP, NCHUNK):
            emit_steady(j // GRP, j % GRP)

        for j in (NCHUNK - 2, NCHUNK - 1):
            rb = (j % GRP) % NBUF
            q = (j % GRP) % NIDX
            pltpu.make_async_copy(rows[rb], acc.at[di[q]], ssem[rb]).wait()
        plsc.subcore_barrier()

        for t in range(NSTRIPE):
            r = row0 + t * ZROWS
            pltpu.sync_copy(acc.at[pl.ds(r, ZROWS)], out_hbm.at[c, pl.ds(r, ZROWS)])

    return k(h, src, dst)


def _combine(p, h, Wr, Wroot, b2d, relu):
    BR = 640

    def body(p_ref, h_ref, wr_ref, wroot_ref, b_ref, o_ref):
        agg = p_ref[0] + p_ref[1]
        out = jnp.dot(agg, wr_ref[...], preferred_element_type=jnp.float32)
        out = out + jnp.dot(h_ref[...], wroot_ref[...],
                            preferred_element_type=jnp.float32)
        out = out + b_ref[...]
        if relu:
            out = jnp.maximum(out, 0.0)
        o_ref[...] = out

    return pl.pallas_call(
        body,
        grid=(NPAD // BR,),
        in_specs=[
            pl.BlockSpec((NC, BR, D), lambda i: (0, i, 0)),
            pl.BlockSpec((BR, D), lambda i: (i, 0)),
            pl.BlockSpec((D, D), lambda i: (0, 0)),
            pl.BlockSpec((D, D), lambda i: (0, 0)),
            pl.BlockSpec((1, D), lambda i: (0, 0)),
        ],
        out_specs=pl.BlockSpec((BR, D), lambda i: (i, 0)),
        out_shape=jax.ShapeDtypeStruct((NPAD, D), jnp.float32),
    )(p, h, Wr, Wroot, b2d)


def kernel(x, edge_index, Wr0, Wroot0, b0, Wr1, Wroot1, b1, Wr2, Wroot2, b2):
    src = edge_index[0].astype(jnp.int32)
    dst = edge_index[1].astype(jnp.int32)
    h = jnp.concatenate([x, jnp.zeros((NPAD - N, D), jnp.float32)], axis=0)
    for Wr, Wroot, b, relu in ((Wr0, Wroot0, b0, True),
                               (Wr1, Wroot1, b1, True),
                               (Wr2, Wroot2, b2, False)):
        p = _seg_sum_partials(h, src, dst)
        h = _combine(p, h, Wr, Wroot, b.reshape(1, D), relu)
    return h[:N]


# speedup vs baseline: 12.7793x; 1.0730x over previous
import functools

import jax
import jax.numpy as jnp
from jax import lax
from jax.experimental import pallas as pl
from jax.experimental.pallas import tpu as pltpu
from jax.experimental.pallas import tpu_sc as plsc

N = 10000
E = 320000
D = 128

NC = 2
NS = 16
NPAD = 10240
CH = 80
EDGES_PER_TILE = E // (NC * NS)
NCHUNK = EDGES_PER_TILE // CH
ROWS_PER_TILE = NPAD // NS
ZROWS = 128
NSTRIPE = ROWS_PER_TILE // ZROWS

NBUF = 4
NIDX = 8
GRP = 8
NGRP = 15


def _seg_sum_partials(h, src, dst):
    mesh = plsc.VectorSubcoreMesh(core_axis_name="c", subcore_axis_name="s")

    @functools.partial(
        pl.kernel,
        mesh=mesh,
        out_type=jax.ShapeDtypeStruct((NC, NPAD, D), jnp.float32),
        scratch_types=(
            [pltpu.VMEM((CH, D), jnp.float32) for _ in range(NBUF)]
            + [pltpu.VMEM((CH,), jnp.int32) for _ in range(NIDX)]
            + [pltpu.VMEM((CH,), jnp.int32) for _ in range(NIDX)]
            + [pltpu.VMEM_SHARED((NPAD, D), jnp.float32)]
            + [pltpu.SemaphoreType.DMA] * (2 * NBUF + 2 * NIDX)
        ),
    )
    def k(h_hbm, src_hbm, dst_hbm, out_hbm, *refs):
        rows = refs[0:NBUF]
        si = refs[NBUF:NBUF + NIDX]
        di = refs[NBUF + NIDX:NBUF + 2 * NIDX]
        acc = refs[NBUF + 2 * NIDX]
        sems = refs[NBUF + 2 * NIDX + 1:]
        gsem = sems[0:NBUF]
        ssem = sems[NBUF:2 * NBUF]
        sisem = sems[2 * NBUF:2 * NBUF + NIDX]
        disem = sems[2 * NBUF + NIDX:]

        c = lax.axis_index("c")
        s = lax.axis_index("s")
        base = c * (E // NC) + s * EDGES_PER_TILE

        def fetch_idx(j, q):
            off = base + j * CH
            pltpu.async_copy(src_hbm.at[pl.ds(off, CH)], si[q], sisem[q])
            pltpu.async_copy(dst_hbm.at[pl.ds(off, CH)], di[q], disem[q])

        def zfill(t, carry):
            i = t // (D // 16)
            j = t % (D // 16)
            rows[0][i, pl.ds(j * 16, 16)] = jnp.zeros((16,), jnp.float32)
            return carry

        lax.fori_loop(0, CH * (D // 16), zfill, 0)

        row0 = s * ROWS_PER_TILE
        for t in range(ROWS_PER_TILE // CH):
            pltpu.async_copy(rows[0], acc.at[pl.ds(row0 + t * CH, CH)], gsem[0])
        for t in range(ROWS_PER_TILE // CH):
            pltpu.make_async_copy(
                rows[0], acc.at[pl.ds(row0 + t * CH, CH)], gsem[0]).wait()
        plsc.subcore_barrier()

        for j in range(6):
            fetch_idx(j, j)
        for j in range(3):
            pltpu.make_async_copy(
                src_hbm.at[pl.ds(base + j * CH, CH)], si[j], sisem[j]).wait()
            pltpu.async_copy(h_hbm.at[si[j]], rows[j], gsem[j])

        def emit_steady(g, b):
            j = g * GRP + b
            static = isinstance(j, int)
            rb = b % NBUF
            rb3 = (b + 3) % NBUF
            q = b % NIDX
            q3 = (b + 3) % NIDX
            q6 = (b + 6) % NIDX
            qp = (b + NIDX - 1) % NIDX

            def fire_fetch():
                fetch_idx(j + 6, q6)

            def wait_scatter_prev():
                pltpu.make_async_copy(
                    rows[rb3], acc.at[di[qp]], ssem[rb3]).wait()

            def fire_gather_next():
                pltpu.make_async_copy(
                    src_hbm.at[pl.ds(base + (j + 3) * CH, CH)],
                    si[q3], sisem[q3]).wait()
                pltpu.async_copy(h_hbm.at[si[q3]], rows[rb3], gsem[rb3])

            if static:
                if j + 6 < NCHUNK:
                    fire_fetch()
                wait_scatter_prev()
                if j + 3 < NCHUNK:
                    fire_gather_next()
            else:
                if b == 7:
                    @pl.when(j + 6 < NCHUNK)
                    def _():
                        fire_fetch()
                else:
                    fire_fetch()
                if b == 0:
                    @pl.when(j >= 1)
                    def _():
                        wait_scatter_prev()
                else:
                    wait_scatter_prev()
                fire_gather_next()
            pltpu.make_async_copy(
                h_hbm.at[si[q]], rows[rb], gsem[rb]).wait()
            pltpu.make_async_copy(
                dst_hbm.at[pl.ds(base + j * CH, CH)], di[q], disem[q]).wait()
            pltpu.async_copy(rows[rb], acc.at[di[q]], ssem[rb], add=True)

        def group(g, carry):
            for b in range(GRP):
                emit_steady(g, b)
            return carry

        lax.fori_loop(0, NGRP, group, 0)

        for j in range(NGRP * GRP, NCHUNK):
            emit_steady(j // GRP, j % GRP)

        jl = NCHUNK - 1
        pltpu.make_async_copy(
            rows[(jl % GRP) % NBUF], acc.at[di[jl % NIDX]],
            ssem[(jl % GRP) % NBUF]).wait()
        plsc.subcore_barrier()

        for t in range(NSTRIPE):
            r = row0 + t * ZROWS
            pltpu.async_copy(acc.at[pl.ds(r, ZROWS)],
                             out_hbm.at[c, pl.ds(r, ZROWS)], gsem[0])
        for t in range(NSTRIPE):
            r = row0 + t * ZROWS
            pltpu.make_async_copy(acc.at[pl.ds(r, ZROWS)],
                                  out_hbm.at[c, pl.ds(r, ZROWS)], gsem[0]).wait()

    return k(h, src, dst)


def _combine(p, h, Wr, Wroot, b2d, relu):
    BR = 640

    def body(p_ref, h_ref, wr_ref, wroot_ref, b_ref, o_ref):
        agg = p_ref[0] + p_ref[1]
        out = jnp.dot(agg, wr_ref[...], preferred_element_type=jnp.float32)
        out = out + jnp.dot(h_ref[...], wroot_ref[...],
                            preferred_element_type=jnp.float32)
        out = out + b_ref[...]
        if relu:
            out = jnp.maximum(out, 0.0)
        o_ref[...] = out

    return pl.pallas_call(
        body,
        grid=(NPAD // BR,),
        in_specs=[
            pl.BlockSpec((NC, BR, D), lambda i: (0, i, 0)),
            pl.BlockSpec((BR, D), lambda i: (i, 0)),
            pl.BlockSpec((D, D), lambda i: (0, 0)),
            pl.BlockSpec((D, D), lambda i: (0, 0)),
            pl.BlockSpec((1, D), lambda i: (0, 0)),
        ],
        out_specs=pl.BlockSpec((BR, D), lambda i: (i, 0)),
        out_shape=jax.ShapeDtypeStruct((NPAD, D), jnp.float32),
    )(p, h, Wr, Wroot, b2d)


def kernel(x, edge_index, Wr0, Wroot0, b0, Wr1, Wroot1, b1, Wr2, Wroot2, b2):
    src = edge_index[0].astype(jnp.int32)
    dst = edge_index[1].astype(jnp.int32)
    h = jnp.concatenate([x, jnp.zeros((NPAD - N, D), jnp.float32)], axis=0)
    for Wr, Wroot, b, relu in ((Wr0, Wroot0, b0, True),
                               (Wr1, Wroot1, b1, True),
                               (Wr2, Wroot2, b2, False)):
        p = _seg_sum_partials(h, src, dst)
        h = _combine(p, h, Wr, Wroot, b.reshape(1, D), relu)
    return h[:N]


# speedup vs baseline: 14.1484x; 1.1071x over previous
import functools

import jax
import jax.numpy as jnp
from jax import lax
from jax.experimental import pallas as pl
from jax.experimental.pallas import tpu as pltpu
from jax.experimental.pallas import tpu_sc as plsc

N = 10000
E = 320000
D = 128

NC = 2
NS = 16
NPAD = 10240
CH = 80
EDGES_PER_TILE = E // (NC * NS)
NCHUNK = EDGES_PER_TILE // CH
ROWS_PER_TILE = NPAD // NS
ZROWS = 128
NSTRIPE = ROWS_PER_TILE // ZROWS

NBUF = 4
NIDX = 8
GRP = 8
NGRP = 15


def _seg_sum_partials(h, edge_index):
    mesh = plsc.VectorSubcoreMesh(core_axis_name="c", subcore_axis_name="s")

    @functools.partial(
        pl.kernel,
        mesh=mesh,
        out_type=jax.ShapeDtypeStruct((NC, NPAD, D), jnp.float32),
        scratch_types=(
            [pltpu.VMEM((CH, D), jnp.float32) for _ in range(NBUF)]
            + [pltpu.VMEM((2, CH), jnp.int32) for _ in range(NIDX)]
            + [pltpu.VMEM_SHARED((NPAD, D), jnp.float32)]
            + [pltpu.SemaphoreType.DMA] * (2 * NBUF + NIDX)
        ),
    )
    def k(h_hbm, ei_hbm, out_hbm, *refs):
        rows = refs[0:NBUF]
        idx = refs[NBUF:NBUF + NIDX]
        acc = refs[NBUF + NIDX]
        sems = refs[NBUF + NIDX + 1:]
        gsem = sems[0:NBUF]
        ssem = sems[NBUF:2 * NBUF]
        isem = sems[2 * NBUF:]

        c = lax.axis_index("c")
        s = lax.axis_index("s")
        base = c * (E // NC) + s * EDGES_PER_TILE

        def fetch_idx(j, q):
            off = base + j * CH
            pltpu.async_copy(ei_hbm.at[pl.ds(off, CH)], idx[q].at[0], isem[q])
            pltpu.async_copy(
                ei_hbm.at[pl.ds(E + off, CH)], idx[q].at[1], isem[q])

        def wait_idx(j, q):
            off = base + j * CH
            pltpu.make_async_copy(
                ei_hbm.at[pl.ds(off, CH)], idx[q].at[0], isem[q]).wait()
            pltpu.make_async_copy(
                ei_hbm.at[pl.ds(E + off, CH)], idx[q].at[1], isem[q]).wait()

        def zfill(t, carry):
            i = t // (D // 16)
            j = t % (D // 16)
            rows[0][i, pl.ds(j * 16, 16)] = jnp.zeros((16,), jnp.float32)
            return carry

        lax.fori_loop(0, CH * (D // 16), zfill, 0)

        row0 = s * ROWS_PER_TILE
        for t in range(ROWS_PER_TILE // CH):
            pltpu.async_copy(rows[0], acc.at[pl.ds(row0 + t * CH, CH)], gsem[0])
        for t in range(ROWS_PER_TILE // CH):
            pltpu.make_async_copy(
                rows[0], acc.at[pl.ds(row0 + t * CH, CH)], gsem[0]).wait()
        plsc.subcore_barrier()

        for j in range(6):
            fetch_idx(j, j)
        for j in range(3):
            wait_idx(j, j)
            pltpu.async_copy(h_hbm.at[idx[j].at[0]], rows[j], gsem[j])

        def emit_steady(g, b):
            j = g * GRP + b
            static = isinstance(j, int)
            rb = b % NBUF
            rb3 = (b + 3) % NBUF
            q = b % NIDX
            q3 = (b + 3) % NIDX
            q6 = (b + 6) % NIDX
            qp = (b + NIDX - 1) % NIDX

            def fire_fetch():
                fetch_idx(j + 6, q6)

            def wait_scatter_prev():
                pltpu.make_async_copy(
                    rows[rb3], acc.at[idx[qp].at[1]], ssem[rb3]).wait()

            def fire_gather_next():
                wait_idx(j + 3, q3)
                pltpu.async_copy(h_hbm.at[idx[q3].at[0]], rows[rb3], gsem[rb3])

            if static:
                if j + 6 < NCHUNK:
                    fire_fetch()
                wait_scatter_prev()
                if j + 3 < NCHUNK:
                    fire_gather_next()
            else:
                if b == 7:
                    @pl.when(j + 6 < NCHUNK)
                    def _():
                        fire_fetch()
                else:
                    fire_fetch()
                if b == 0:
                    @pl.when(j >= 1)
                    def _():
                        wait_scatter_prev()
                else:
                    wait_scatter_prev()
                fire_gather_next()
            pltpu.make_async_copy(
                h_hbm.at[idx[q].at[0]], rows[rb], gsem[rb]).wait()
            pltpu.async_copy(rows[rb], acc.at[idx[q].at[1]], ssem[rb], add=True)

        def group(g, carry):
            for b in range(GRP):
                emit_steady(g, b)
            return carry

        lax.fori_loop(0, NGRP, group, 0)

        for j in range(NGRP * GRP, NCHUNK):
            emit_steady(j // GRP, j % GRP)

        jl = NCHUNK - 1
        pltpu.make_async_copy(
            rows[(jl % GRP) % NBUF], acc.at[idx[jl % NIDX].at[1]],
            ssem[(jl % GRP) % NBUF]).wait()
        plsc.subcore_barrier()

        for t in range(NSTRIPE):
            r = row0 + t * ZROWS
            pltpu.async_copy(acc.at[pl.ds(r, ZROWS)],
                             out_hbm.at[c, pl.ds(r, ZROWS)], gsem[0])
        for t in range(NSTRIPE):
            r = row0 + t * ZROWS
            pltpu.make_async_copy(acc.at[pl.ds(r, ZROWS)],
                                  out_hbm.at[c, pl.ds(r, ZROWS)], gsem[0]).wait()

    return k(h, edge_index)


def _combine(p, h, Wr, Wroot, b2d, relu):
    BR = 2000

    def body(p_ref, h_ref, wr_ref, wroot_ref, b_ref, o_ref):
        agg = p_ref[0] + p_ref[1]
        out = jnp.dot(agg, wr_ref[...], preferred_element_type=jnp.float32)
        out = out + jnp.dot(h_ref[...], wroot_ref[...],
                            preferred_element_type=jnp.float32)
        out = out + b_ref[...]
        if relu:
            out = jnp.maximum(out, 0.0)
        o_ref[...] = out

    return pl.pallas_call(
        body,
        grid=(N // BR,),
        in_specs=[
            pl.BlockSpec((NC, BR, D), lambda i: (0, i, 0)),
            pl.BlockSpec((BR, D), lambda i: (i, 0)),
            pl.BlockSpec((D, D), lambda i: (0, 0)),
            pl.BlockSpec((D, D), lambda i: (0, 0)),
            pl.BlockSpec((1, D), lambda i: (0, 0)),
        ],
        out_specs=pl.BlockSpec((BR, D), lambda i: (i, 0)),
        out_shape=jax.ShapeDtypeStruct((N, D), jnp.float32),
    )(p, h, Wr, Wroot, b2d)


def kernel(x, edge_index, Wr0, Wroot0, b0, Wr1, Wroot1, b1, Wr2, Wroot2, b2):
    ei = edge_index.astype(jnp.int32).reshape(2 * E)
    h = x
    for Wr, Wroot, b, relu in ((Wr0, Wroot0, b0, True),
                               (Wr1, Wroot1, b1, True),
                               (Wr2, Wroot2, b2, False)):
        p = _seg_sum_partials(h, ei)
        h = _combine(p, h, Wr, Wroot, b.reshape(1, D), relu)
    return h
